# baseline (speedup 1.0000x reference)
import functools

import jax
import jax.numpy as jnp
from jax import lax
from jax.experimental import pallas as pl
from jax.experimental.pallas import tpu as pltpu
from jax.experimental.pallas import tpu_sc as plsc

_NC = 2
_NS = 16
_NW = _NC * _NS
_L = 16


def _off_max_body(x_ref, off_ref):
    x = x_ref[...]
    m = x.shape[-1]
    row = jax.lax.broadcasted_iota(jnp.int32, (m, m), 0)
    col = jax.lax.broadcasted_iota(jnp.int32, (m, m), 1)
    eq = (row == col)[None]
    off_ref[:, 0, 0] = jnp.max(jnp.where(eq, -jnp.inf, x), axis=(1, 2))


def _diag_sc_body(x_hbm, idx_hbm, out_hbm, idx_v, vals_v, out_v, sem):
    wid = lax.axis_index("s") * _NC + lax.axis_index("c")
    chunks = idx_v.shape[0]
    mats_per_w = out_v.shape[0]
    per_mat = chunks // mats_per_w
    pltpu.sync_copy(idx_hbm.at[wid], idx_v)
    copies = [
        pltpu.async_copy(x_hbm.at[idx_v.at[j]], vals_v.at[j], sem)
        for j in range(chunks)
    ]
    for c in copies:
        c.wait()
    for mat in range(mats_per_w):
        acc = jnp.full((_L,), -jnp.inf, dtype=jnp.float32)
        for j in range(mat * per_mat, (mat + 1) * per_mat):
            for k in range(128 // _L):
                acc = jnp.maximum(acc, vals_v[j, pl.ds(k * _L, _L)])
        out_v[mat] = acc
    pltpu.sync_copy(out_v, out_hbm.at[wid])


def _diag_indices(n_mat, m):
    mats_per_w = n_mat // _NW
    mat = jnp.arange(n_mat, dtype=jnp.int32).reshape(_NW, mats_per_w, 1)
    k = jnp.arange(m, dtype=jnp.int32).reshape(1, 1, m)
    idx = mat * (m * m) + k * (m + 1)
    return idx.reshape(_NW, mats_per_w * m // 128, 128)


def kernel(x):
    B, C, m, _ = x.shape
    n_mat = B * C
    mats_per_w = n_mat // _NW
    chunks = mats_per_w * m // 128

    mesh = plsc.VectorSubcoreMesh(core_axis_name="c", subcore_axis_name="s")
    diag_part = functools.partial(
        pl.kernel,
        mesh=mesh,
        out_type=jax.ShapeDtypeStruct((_NW, mats_per_w, _L), jnp.float32),
        scratch_types=[
            pltpu.VMEM((chunks, 128), jnp.int32),
            pltpu.VMEM((chunks, 128), jnp.float32),
            pltpu.VMEM((mats_per_w, _L), jnp.float32),
            pltpu.SemaphoreType.DMA,
        ],
    )(_diag_sc_body)(
        x.reshape(n_mat * m * m),
        _diag_indices(n_mat, m),
    )
    diag = jnp.max(diag_part, axis=-1).reshape(B, C)

    N = 8
    off = pl.pallas_call(
        _off_max_body,
        grid=(n_mat // N,),
        in_specs=[pl.BlockSpec((N, m, m), lambda i: (i, 0, 0))],
        out_specs=pl.BlockSpec((N, 1, 1), lambda i: (i, 0, 0)),
        out_shape=jax.ShapeDtypeStruct((n_mat, 1, 1), x.dtype),
    )(x.reshape(n_mat, m, m))

    return jnp.concatenate((diag, off.reshape(B, C)), axis=-1)


# speedup vs baseline: 3.8034x; 3.8034x over previous
import jax
import jax.numpy as jnp
from jax.experimental import pallas as pl


def _maxes_body(x_ref, out_ref):
    i = pl.program_id(0)
    x = x_ref[...]
    N, m, _ = x.shape
    C2 = out_ref.shape[1]
    C = C2 // 2
    per_row = C // N
    row = jax.lax.broadcasted_iota(jnp.int32, (m, m), 0)
    col = jax.lax.broadcasted_iota(jnp.int32, (m, m), 1)
    eq = (row == col)[None]
    neg = jnp.float32(-jnp.inf)
    dmax = jnp.max(jnp.where(eq, x, neg), axis=(1, 2)).reshape(1, N)
    omax = jnp.max(jnp.where(eq, neg, x), axis=(1, 2)).reshape(1, N)
    n_steps = pl.num_programs(0)
    for step in range(n_steps):
        b = step // per_row
        c0 = (step % per_row) * N

        @pl.when(i == step)
        def _(b=b, c0=c0):
            out_ref[b : b + 1, c0 : c0 + N] = dmax
            out_ref[b : b + 1, C + c0 : C + c0 + N] = omax


def kernel(x):
    B, C, m, _ = x.shape
    n_mat = B * C
    N = 8
    return pl.pallas_call(
        _maxes_body,
        grid=(n_mat // N,),
        in_specs=[pl.BlockSpec((N, m, m), lambda i: (i, 0, 0))],
        out_specs=pl.BlockSpec((B, 2 * C), lambda i: (0, 0)),
        out_shape=jax.ShapeDtypeStruct((B, 2 * C), x.dtype),
    )(x.reshape(n_mat, m, m))
